# baseline (speedup 1.0000x reference)
import functools

import jax
import jax.numpy as jnp
from jax import lax
from jax.experimental import pallas as pl
from jax.experimental.pallas import tpu as pltpu
from jax.experimental.pallas import tpu_sc as plsc

L = 16
NC = 2
NS = 16
NW = NC * NS
B = 4096 * 200
D = 64
BPW = B // NW
C = 512
NCHUNK = BPW // C
SCALE = 8.0

_mesh = plsc.VectorSubcoreMesh(core_axis_name="c", subcore_axis_name="s")


@functools.partial(
    pl.kernel,
    mesh=_mesh,
    out_type=jax.ShapeDtypeStruct((B, D), jnp.float32),
    scratch_types=[
        pltpu.VMEM((C,), jnp.int32),
        pltpu.VMEM((C, D), jnp.float32),
        pltpu.SemaphoreType.DMA,
    ],
    compiler_params=pltpu.CompilerParams(use_tc_tiling_on_sc=False),
)
def _embed_lookup(table_hbm, idx_hbm, out_hbm, idx_v, rows_v, sem):
    wid = lax.axis_index("s") * NC + lax.axis_index("c")
    base = wid * BPW

    def chunk_body(ci, carry):
        off = base + ci * C
        pltpu.sync_copy(idx_hbm.at[pl.ds(off, C)], idx_v)
        pltpu.async_copy(table_hbm.at[idx_v], rows_v, sem).wait()

        def row_body(r, c2):
            for j in range(D // L):
                sl = pl.ds(j * L, L)
                rows_v[r, sl] = rows_v[r, sl] * SCALE
            return c2

        lax.fori_loop(0, C, row_body, 0)
        pltpu.sync_copy(rows_v, out_hbm.at[pl.ds(off, C)])
        return carry

    lax.fori_loop(0, NCHUNK, chunk_body, 0)


def kernel(token_ids, embed_table):
    idx = token_ids.reshape(-1)
    out = _embed_lookup(embed_table, idx)
    return out.reshape(token_ids.shape[0], token_ids.shape[1], D)


# speedup vs baseline: 1.1355x; 1.1355x over previous
import functools

import jax
import jax.numpy as jnp
from jax import lax
from jax.experimental import pallas as pl
from jax.experimental.pallas import tpu as pltpu
from jax.experimental.pallas import tpu_sc as plsc

L = 16
NC = 2
NS = 16
NW = NC * NS
B = 4096 * 200
D = 64
BPW = B // NW
C = 512
NCHUNK = BPW // C
SCALE = 8.0

_mesh = plsc.VectorSubcoreMesh(core_axis_name="c", subcore_axis_name="s")


@functools.partial(
    pl.kernel,
    mesh=_mesh,
    out_type=jax.ShapeDtypeStruct((B, D), jnp.float32),
    scratch_types=[
        pltpu.VMEM((BPW,), jnp.int32),
        pltpu.VMEM((C, D), jnp.float32),
        pltpu.VMEM((C, D), jnp.float32),
        pltpu.SemaphoreType.DMA,
        pltpu.SemaphoreType.DMA,
        pltpu.SemaphoreType.DMA,
        pltpu.SemaphoreType.DMA,
    ],
    compiler_params=pltpu.CompilerParams(use_tc_tiling_on_sc=False),
)
def _embed_lookup(table_hbm, idx_hbm, out_hbm, idx_v, rows0, rows1,
                  g0, g1, o0, o1):
    wid = lax.axis_index("s") * NC + lax.axis_index("c")
    base = wid * BPW
    rows = (rows0, rows1)
    gsem = (g0, g1)
    osem = (o0, o1)

    pltpu.sync_copy(idx_hbm.at[pl.ds(base, BPW)], idx_v)

    def start_gather(ci, b):
        pltpu.async_copy(
            table_hbm.at[idx_v.at[pl.ds(ci * C, C)]], rows[b], gsem[b])

    def wait_gather(ci, b):
        pltpu.make_async_copy(
            table_hbm.at[idx_v.at[pl.ds(ci * C, C)]], rows[b], gsem[b]).wait()

    def wait_out(ci, b):
        pltpu.make_async_copy(
            rows[b], out_hbm.at[pl.ds(base + ci * C, C)], osem[b]).wait()

    def scale_chunk(b):
        def row_body(r2, carry):
            for u in range(2):
                r = r2 * 2 + u
                for j in range(D // L):
                    sl = pl.ds(j * L, L)
                    rows[b][r, sl] = rows[b][r, sl] * SCALE
            return carry
        lax.fori_loop(0, C // 2, row_body, 0)

    def start_out(ci, b):
        pltpu.async_copy(rows[b], out_hbm.at[pl.ds(base + ci * C, C)],
                         osem[b])

    start_gather(0, 0)
    start_gather(1, 1)

    wait_gather(0, 0)
    scale_chunk(0)
    start_out(0, 0)
    wait_gather(1, 1)
    scale_chunk(1)
    start_out(1, 1)

    def superstep(k, carry):
        ci0 = 2 * k
        ci1 = 2 * k + 1
        wait_out(ci0, 0)
        start_gather(ci0, 0)
        wait_out(ci1, 1)
        start_gather(ci1, 1)
        wait_gather(ci0, 0)
        scale_chunk(0)
        start_out(ci0, 0)
        wait_gather(ci1, 1)
        scale_chunk(1)
        start_out(ci1, 1)
        return carry

    lax.fori_loop(1, NCHUNK // 2, superstep, 0)
    wait_out(NCHUNK - 2, 0)
    wait_out(NCHUNK - 1, 1)


def kernel(token_ids, embed_table):
    idx = token_ids.reshape(-1)
    out = _embed_lookup(embed_table, idx)
    return out.reshape(token_ids.shape[0], token_ids.shape[1], D)
